# baseline (speedup 1.0000x reference)
import functools

import jax
import jax.numpy as jnp
from jax import lax
from jax.experimental import pallas as pl
from jax.experimental.pallas import tpu as pltpu
from jax.experimental.pallas import tpu_sc as plsc

_NC = 2
_NS = 16
_NW = _NC * _NS
_L = 16
_BB = 128
_EMB = 64
_RS = 64
_VB = 320


def _sc_table_relayout(t64):
    emb, vocab = t64.shape
    nblk = vocab // _VB
    iters = (nblk + _NW - 1) // _NW
    last_w = nblk - (iters - 1) * _NW

    mesh = plsc.VectorSubcoreMesh(core_axis_name="c", subcore_axis_name="s")

    @functools.partial(
        pl.kernel,
        mesh=mesh,
        out_type=jax.ShapeDtypeStruct((vocab, _RS), jnp.float32),
        scratch_types=[
            pltpu.VMEM((2, _EMB, _VB), jnp.float32),
            pltpu.VMEM((2, _VB, _RS + 1), jnp.float32),
            pltpu.SemaphoreType.DMA,
            pltpu.SemaphoreType.DMA,
            pltpu.SemaphoreType.DMA,
            pltpu.SemaphoreType.DMA,
        ],
        compiler_params=pltpu.CompilerParams(
            use_tc_tiling_on_sc=False, needs_layout_passes=False
        ),
    )
    def body(t_hbm, scr_hbm, in_v, out_v, a0, a1, b0, b1):
        wid = lax.axis_index("s") * _NC + lax.axis_index("c")
        asems = (a0, a1)
        bsems = (b0, b1)

        def v0_of(it):
            return (wid + _NW * it) * _VB

        def start_in(it, buf):
            pltpu.async_copy(
                t_hbm.at[pl.ds(0, _EMB), pl.ds(v0_of(it), _VB)],
                in_v.at[buf],
                asems[buf],
            )

        def wait_in(buf):
            pltpu.make_async_copy(
                t_hbm.at[pl.ds(0, _EMB), pl.ds(0, _VB)], in_v.at[buf],
                asems[buf],
            ).wait()

        def start_out(it, buf):
            pltpu.async_copy(
                out_v.at[buf, pl.ds(0, _VB), pl.ds(0, _RS)],
                scr_hbm.at[pl.ds(v0_of(it), _VB)],
                bsems[buf],
            )

        def wait_out(buf):
            pltpu.make_async_copy(
                out_v.at[buf, pl.ds(0, _VB), pl.ds(0, _RS)],
                scr_hbm.at[pl.ds(0, _VB)],
                bsems[buf],
            ).wait()

        def transpose(buf):
            iota = lax.iota(jnp.int32, _L)
            dvecs = [iota + (j * _L) for j in range(_VB // _L)]
            ob = out_v.at[buf]

            def tbody(c, cv):
                for j in range(_VB // _L):
                    vals = in_v[buf, c, pl.ds(j * _L, _L)]
                    plsc.store_scatter(ob, [dvecs[j], cv], vals)
                return cv + 1

            plsc.parallel_loop(
                0, _EMB, step=1, unroll=8, carry=jnp.zeros((_L,), jnp.int32)
            )(tbody)

        start_in(0, 0)

        wait_in(0)
        start_in(1, 1)
        transpose(0)
        start_out(0, 0)

        wait_in(1)
        start_in(2, 0)
        transpose(1)
        start_out(1, 1)

        def steady(p, carry):
            it = 2 * p + 2
            wait_in(0)
            start_in(it + 1, 1)
            wait_out(0)
            transpose(0)
            start_out(it, 0)

            wait_in(1)
            start_in(it + 2, 0)
            wait_out(1)
            transpose(1)
            start_out(it + 1, 1)
            return carry

        lax.fori_loop(0, (iters - 4) // 2, steady, 0)

        it = iters - 2
        wait_in(0)

        @pl.when(wid < last_w)
        def _():
            start_in(it + 1, 1)

        wait_out(0)
        transpose(0)
        start_out(it, 0)

        @pl.when(wid < last_w)
        def _():
            wait_in(1)
            wait_out(1)
            transpose(1)
            start_out(it + 1, 1)

        wait_out(0)
        wait_out(1)

    return body(t64)


def _sc_embedding_lookup(ids4, scr, b, s):
    nbt = b // _BB
    nst = s // 8
    assert nbt == _NW and nst * 8 == s and s % 2 == 0

    mesh = plsc.VectorSubcoreMesh(core_axis_name="c", subcore_axis_name="s")

    @functools.partial(
        pl.kernel,
        mesh=mesh,
        out_type=jax.ShapeDtypeStruct((s, _EMB // 8, nbt, 8, _BB), jnp.float32),
        scratch_types=[
            pltpu.VMEM((nst, 8, _BB), jnp.int32),
            pltpu.VMEM((2, _BB, _RS), jnp.float32),
            pltpu.VMEM((2, _EMB, _BB + 1), jnp.float32),
            pltpu.SemaphoreType.DMA,
            pltpu.SemaphoreType.DMA,
            pltpu.SemaphoreType.DMA,
            pltpu.SemaphoreType.DMA,
        ],
        compiler_params=pltpu.CompilerParams(
            use_tc_tiling_on_sc=False, needs_layout_passes=False
        ),
    )
    def body(ids_hbm, scr_hbm, y_hbm, idx_v, rows_v, yblk_v, g0, g1, s0, s1):
        wid = lax.axis_index("s") * _NC + lax.axis_index("c")
        gsems = (g0, g1)
        ssems = (s0, s1)

        def start_gather(k, buf):
            st = k // 8
            sr = k % 8
            pltpu.async_copy(
                scr_hbm.at[idx_v.at[st, sr]], rows_v.at[buf], gsems[buf]
            )

        def wait_gather(buf):
            pltpu.make_async_copy(
                scr_hbm.at[pl.ds(0, _BB)], rows_v.at[buf], gsems[buf]
            ).wait()

        def start_store(k, buf):
            for tc in range(_EMB // 8):
                pltpu.async_copy(
                    yblk_v.at[buf, pl.ds(tc * 8, 8), pl.ds(0, _BB)],
                    y_hbm.at[k, tc, wid],
                    ssems[buf],
                )

        def wait_store(buf):
            for tc in range(_EMB // 8):
                pltpu.make_async_copy(
                    yblk_v.at[buf, pl.ds(tc * 8, 8), pl.ds(0, _BB)],
                    y_hbm.at[0, tc, 0],
                    ssems[buf],
                ).wait()

        def transpose(buf):
            iota = lax.iota(jnp.int32, _L)
            cvecs = [iota + (j * _L) for j in range(_EMB // _L)]
            yb = yblk_v.at[buf]

            def tbody(br, brv):
                for j in range(_EMB // _L):
                    vals = rows_v[buf, br, pl.ds(j * _L, _L)]
                    plsc.store_scatter(yb, [cvecs[j], brv], vals)
                return brv + 1

            plsc.parallel_loop(
                0, _BB, step=1, unroll=8, carry=jnp.zeros((_L,), jnp.int32)
            )(tbody)

        for st in range(nst):
            pltpu.sync_copy(ids_hbm.at[st, wid], idx_v.at[st])

        nblk = s
        start_gather(0, 0)

        wait_gather(0)
        start_gather(1, 1)
        transpose(0)
        start_store(0, 0)

        wait_gather(1)
        start_gather(2, 0)
        transpose(1)
        start_store(1, 1)

        def steady(p, carry):
            k = 2 * p + 2
            wait_gather(0)
            start_gather(k + 1, 1)
            wait_store(0)
            transpose(0)
            start_store(k, 0)

            wait_gather(1)
            start_gather(k + 2, 0)
            wait_store(1)
            transpose(1)
            start_store(k + 1, 1)
            return carry

        lax.fori_loop(0, (nblk - 4) // 2, steady, 0)

        wait_gather(0)
        start_gather(nblk - 1, 1)
        wait_store(0)
        transpose(0)
        start_store(nblk - 2, 0)

        wait_gather(1)
        wait_store(1)
        transpose(1)
        start_store(nblk - 1, 1)

        wait_store(0)
        wait_store(1)

    return body(ids4, scr)


def kernel(token_ids, table):
    b, s = token_ids.shape
    emb = table.shape[1]
    t64 = table.T
    scr = _sc_table_relayout(t64)
    ids4 = token_ids.T.reshape(s // 8, 8, b // 128, 128).transpose(0, 2, 1, 3)
    y5 = _sc_embedding_lookup(ids4, scr, b, s)
    return y5.transpose(2, 4, 0, 1, 3).reshape(b, s, emb)


# speedup vs baseline: 6.6346x; 6.6346x over previous
import functools

import jax
import jax.numpy as jnp
from jax import lax
from jax.experimental import pallas as pl
from jax.experimental.pallas import tpu as pltpu
from jax.experimental.pallas import tpu_sc as plsc

_NC = 2
_NS = 16
_NW = _NC * _NS
_L = 16
_BB = 128
_EMB = 64


def _sc_embedding_lookup(ids4, table, b, s):
    nbt = b // _BB
    nst = s // 8
    assert nbt == _NW and nst * 8 == s and s % 2 == 0

    mesh = plsc.VectorSubcoreMesh(core_axis_name="c", subcore_axis_name="s")

    @functools.partial(
        pl.kernel,
        mesh=mesh,
        out_type=jax.ShapeDtypeStruct((s, _EMB // 8, nbt, 8, _BB), jnp.float32),
        scratch_types=[
            pltpu.VMEM((nst, 8, _BB), jnp.int32),
            pltpu.VMEM((2, _BB, _EMB), jnp.float32),
            pltpu.VMEM((2, _EMB, _BB + 1), jnp.float32),
            pltpu.SemaphoreType.DMA,
            pltpu.SemaphoreType.DMA,
            pltpu.SemaphoreType.DMA,
            pltpu.SemaphoreType.DMA,
        ],
        compiler_params=pltpu.CompilerParams(
            use_tc_tiling_on_sc=False, needs_layout_passes=False
        ),
    )
    def body(ids_hbm, table_hbm, y_hbm, idx_v, rows_v, yblk_v, g0, g1, s0, s1):
        wid = lax.axis_index("s") * _NC + lax.axis_index("c")
        gsems = (g0, g1)
        ssems = (s0, s1)

        def start_gather(k, buf):
            st = k // 8
            sr = k % 8
            pltpu.async_copy(
                table_hbm.at[idx_v.at[st, sr]], rows_v.at[buf], gsems[buf]
            )

        def wait_gather(buf):
            pltpu.make_async_copy(
                table_hbm.at[pl.ds(0, _BB)], rows_v.at[buf], gsems[buf]
            ).wait()

        def start_store(k, buf):
            for tc in range(_EMB // 8):
                pltpu.async_copy(
                    yblk_v.at[buf, pl.ds(tc * 8, 8), pl.ds(0, _BB)],
                    y_hbm.at[k, tc, wid],
                    ssems[buf],
                )

        def wait_store(buf):
            for tc in range(_EMB // 8):
                pltpu.make_async_copy(
                    yblk_v.at[buf, pl.ds(tc * 8, 8), pl.ds(0, _BB)],
                    y_hbm.at[0, tc, 0],
                    ssems[buf],
                ).wait()

        def transpose(buf):
            iota = lax.iota(jnp.int32, _L)
            cvecs = [iota + (j * _L) for j in range(_EMB // _L)]
            yb = yblk_v.at[buf]

            def tbody(br, brv):
                for j in range(_EMB // _L):
                    vals = rows_v[buf, br, pl.ds(j * _L, _L)]
                    plsc.store_scatter(yb, [cvecs[j], brv], vals)
                return brv + 1

            plsc.parallel_loop(
                0, _BB, step=1, unroll=8, carry=jnp.zeros((_L,), jnp.int32)
            )(tbody)

        id_copies = [
            pltpu.async_copy(ids_hbm.at[st, wid], idx_v.at[st], g0)
            for st in range(nst)
        ]
        for c in id_copies:
            c.wait()

        nblk = s
        start_gather(0, 0)

        wait_gather(0)
        start_gather(1, 1)
        transpose(0)
        start_store(0, 0)

        wait_gather(1)
        start_gather(2, 0)
        transpose(1)
        start_store(1, 1)

        def steady(p, carry):
            k = 2 * p + 2
            wait_gather(0)
            start_gather(k + 1, 1)
            wait_store(0)
            transpose(0)
            start_store(k, 0)

            wait_gather(1)
            start_gather(k + 2, 0)
            wait_store(1)
            transpose(1)
            start_store(k + 1, 1)
            return carry

        lax.fori_loop(0, (nblk - 4) // 2, steady, 0)

        wait_gather(0)
        start_gather(nblk - 1, 1)
        wait_store(0)
        transpose(0)
        start_store(nblk - 2, 0)

        wait_gather(1)
        wait_store(1)
        transpose(1)
        start_store(nblk - 1, 1)

        wait_store(0)
        wait_store(1)

    return body(ids4, table)


def kernel(token_ids, table):
    b, s = token_ids.shape
    emb = table.shape[1]
    ids4 = token_ids.T.reshape(s // 8, 8, b // 128, 128).transpose(0, 2, 1, 3)
    y5 = _sc_embedding_lookup(ids4, table, b, s)
    return y5.transpose(2, 4, 0, 1, 3).reshape(b, s, emb)
